# Initial kernel scaffold; baseline (speedup 1.0000x reference)
import jax
import jax.numpy as jnp
from jax.experimental import pallas as pl


def kernel(bezier_proj_centerline_img, ref_catheter_centerline):
    raise NotImplementedError("write your pallas kernel here")


# speedup vs baseline: 1.4374x; 1.4374x over previous
import functools

import jax
import jax.numpy as jnp
from jax.experimental import pallas as pl
from jax.experimental.pallas import tpu as pltpu

N = 8192
M = 8192
BN = 512
BM = 512
BIG = 3.0e38


def _tc_body(bx_ref, by_ref, rx_ref, ry_ref, out_ref,
             rowacc, colacc, sum1, cnt, sum2):
    i = pl.program_id(0)
    j = pl.program_id(1)
    ni = pl.num_programs(0)
    nj = pl.num_programs(1)

    bx = bx_ref[...]
    by = by_ref[...]
    rx = rx_ref[...]
    ry = ry_ref[...]

    dx = bx - rx
    dy = by - ry
    d2 = dx * dx + dy * dy

    mask = ((bx >= -2000.0) & (bx <= 2000.0) &
            (by >= -2000.0) & (by <= 2000.0))

    rmin = jnp.min(d2, axis=1, keepdims=True)

    @pl.when(j == 0)
    def _():
        rowacc[...] = rmin

    @pl.when(j != 0)
    def _():
        rowacc[...] = jnp.minimum(rowacc[...], rmin)

    d2m = jnp.where(mask, d2, BIG)
    cmin = jnp.min(d2m, axis=0, keepdims=True)

    @pl.when(i == 0)
    def _():
        colacc[0:1, pl.ds(j * BM, BM)] = cmin

    @pl.when(i != 0)
    def _():
        colacc[0:1, pl.ds(j * BM, BM)] = jnp.minimum(
            colacc[0:1, pl.ds(j * BM, BM)], cmin)

    @pl.when((i == 0) & (j == 0))
    def _():
        sum1[0] = 0.0
        cnt[0] = 0.0
        sum2[0] = 0.0

    @pl.when(j == nj - 1)
    def _():
        rd = jnp.sqrt(rowacc[...])
        maskf = mask.astype(jnp.float32)
        sum1[0] += jnp.sum(rd * maskf)
        cnt[0] += jnp.sum(maskf)

    @pl.when(i == ni - 1)
    def _():
        cd = jnp.sqrt(colacc[0:1, pl.ds(j * BM, BM)])
        sum2[0] += jnp.sum(cd)

    @pl.when((i == ni - 1) & (j == nj - 1))
    def _():
        mean1 = sum1[0] / jnp.maximum(cnt[0], 1.0)
        mean2 = sum2[0] / jnp.float32(M)
        out_ref[0, 0] = (mean1 + mean2) * 0.5


@jax.jit
def _centerline_loss_tc(bez, ref):
    bx = bez[:, 0:1]
    by = bez[:, 1:2]
    rx = ref[:, 1].reshape(1, M)
    ry = ref[:, 0].reshape(1, M)

    grid = (N // BN, M // BM)
    out = pl.pallas_call(
        _tc_body,
        grid=grid,
        in_specs=[
            pl.BlockSpec((BN, 1), lambda i, j: (i, 0)),
            pl.BlockSpec((BN, 1), lambda i, j: (i, 0)),
            pl.BlockSpec((1, BM), lambda i, j: (0, j)),
            pl.BlockSpec((1, BM), lambda i, j: (0, j)),
        ],
        out_specs=pl.BlockSpec(memory_space=pltpu.SMEM),
        out_shape=jax.ShapeDtypeStruct((1, 1), jnp.float32),
        scratch_shapes=[
            pltpu.VMEM((BN, 1), jnp.float32),
            pltpu.VMEM((1, M), jnp.float32),
            pltpu.SMEM((1,), jnp.float32),
            pltpu.SMEM((1,), jnp.float32),
            pltpu.SMEM((1,), jnp.float32),
        ],
    )(bx, by, rx, ry)
    return out[0, 0]


def kernel(bezier_proj_centerline_img, ref_catheter_centerline):
    return _centerline_loss_tc(bezier_proj_centerline_img,
                               ref_catheter_centerline)
